# baseline (speedup 1.0000x reference)
import functools

import jax
import jax.numpy as jnp
from jax import lax
from jax.experimental import pallas as pl
from jax.experimental.pallas import tpu as pltpu
from jax.experimental.pallas import tpu_sc as plsc

NN = 10000
NP = 10240
DD = 256
NE = 160000
NW = 32
EW = 10240
ED = NW * EW
NB = 4
BR = NP // NB
CAP = EW
ZR = 32
DR = 128
KC = 64
STG = 4
NTH = EW // STG // KC
KX = 32
SCH = 20
BM = 512
GB = NP // BM


def _make_spmm_deg():
    mesh = plsc.VectorSubcoreMesh(core_axis_name="c", subcore_axis_name="s")
    out_type = [jax.ShapeDtypeStruct((NP, 128), jnp.float32),
                jax.ShapeDtypeStruct((NP, 128), jnp.float32)]
    scratch = [
        pltpu.VMEM((NTH, KC), jnp.int32),
        pltpu.VMEM((NTH, KC), jnp.int32),
        pltpu.VMEM((KC, 128), jnp.float32),
        pltpu.VMEM((KC, 128), jnp.float32),
        pltpu.VMEM((KC, 128), jnp.float32),
        pltpu.VMEM((KC, 128), jnp.float32),
        pltpu.VMEM_SHARED((NP, 128), jnp.float32),
        pltpu.SemaphoreType.DMA,
        pltpu.SemaphoreType.DMA,
    ]

    @functools.partial(pl.kernel, mesh=mesh, out_type=out_type,
                       scratch_types=scratch,
                       compiler_params=pltpu.CompilerParams(
                           use_tc_tiling_on_sc=True))
    def spmm(rows_hbm, cols_hbm, zrow_hbm, g, out0, out1,
             colbuf, rowbuf, b0, b1, b2, b3, acc, sem_g, sem_s):
        bufs = (b0, b1, b2, b3)
        c = lax.axis_index("c")
        s = lax.axis_index("s")
        wid = s * 2 + c
        rps = NP // 16

        def zloop(i, _):
            pltpu.sync_copy(zrow_hbm, acc.at[pl.ds(s * rps + i * ZR, ZR)])
            return 0
        lax.fori_loop(0, rps // ZR, zloop, 0)
        plsc.subcore_barrier()

        for h in range(STG):
            base = wid * STG * NTH + h * NTH
            pltpu.sync_copy(cols_hbm.at[pl.ds(base, NTH)], colbuf)
            pltpu.sync_copy(rows_hbm.at[pl.ds(base, NTH)], rowbuf)
            pltpu.async_copy(g.at[colbuf.at[0]], b0, sem_g)
            pltpu.async_copy(g.at[colbuf.at[1]], b1, sem_g)

            def group(i, _):
                for b in range(4):
                    t = 4 * i + b
                    cur = bufs[b]
                    nxt = bufs[(b + 2) % 4]

                    @pl.when(t >= 2)
                    def _():
                        pltpu.make_async_copy(
                            nxt, acc.at[rowbuf.at[t - 2]], sem_s).wait()

                    @pl.when(t + 2 < NTH)
                    def _():
                        pltpu.async_copy(g.at[colbuf.at[t + 2]], nxt, sem_g)

                    pltpu.make_async_copy(
                        g.at[colbuf.at[t]], cur, sem_g).wait()
                    pltpu.async_copy(
                        cur, acc.at[rowbuf.at[t]], sem_s, add=True)
                return 0
            lax.fori_loop(0, NTH // 4, group, 0)
            pltpu.make_async_copy(
                bufs[(NTH - 2) % 4], acc.at[rowbuf.at[NTH - 2]], sem_s).wait()
            pltpu.make_async_copy(
                bufs[(NTH - 1) % 4], acc.at[rowbuf.at[NTH - 1]], sem_s).wait()
        plsc.subcore_barrier()

        def dloop(i, _):
            r0 = s * rps + i * DR

            @pl.when(c == 0)
            def _():
                pltpu.sync_copy(acc.at[pl.ds(r0, DR)], out0.at[pl.ds(r0, DR)])

            @pl.when(c == 1)
            def _():
                pltpu.sync_copy(acc.at[pl.ds(r0, DR)], out1.at[pl.ds(r0, DR)])
            return 0
        lax.fori_loop(0, rps // DR, dloop, 0)

    return spmm


def _make_binner():
    mesh = plsc.VectorSubcoreMesh(core_axis_name="c", subcore_axis_name="s")
    out_type = [jax.ShapeDtypeStruct((NW * NB * CAP,), jnp.int32),
                jax.ShapeDtypeStruct((NW * NB, 16), jnp.int32)]
    scratch = [
        pltpu.VMEM((EW // STG,), jnp.int32),
        pltpu.VMEM((EW // STG,), jnp.int32),
        pltpu.VMEM((NB * CAP,), jnp.int32),
        pltpu.VMEM((NB, 16), jnp.int32),
    ]

    @functools.partial(pl.kernel, mesh=mesh, out_type=out_type,
                       scratch_types=scratch,
                       compiler_params=pltpu.CompilerParams(
                           use_tc_tiling_on_sc=True,
                           needs_layout_passes=False))
    def binner(rows_hbm, cols_hbm, bins, counts, rbuf, cbuf, binflat, cntbuf):
        c = lax.axis_index("c")
        s = lax.axis_index("s")
        wid = s * 2 + c
        sl = EW // STG
        filler = jnp.full((16,), NN, jnp.int32)

        def floop(v, _):
            binflat[pl.ds(v * 16, 16)] = filler
            return 0
        lax.fori_loop(0, NB * CAP // 16, floop, 0)

        def stage_body(st, curs):
            pltpu.sync_copy(rows_hbm.at[pl.ds(wid * EW + st * sl, sl)], rbuf)
            pltpu.sync_copy(cols_hbm.at[pl.ds(wid * EW + st * sl, sl)], cbuf)

            def vloop(v, curs):
                row = rbuf[pl.ds(v * 16, 16)]
                col = cbuf[pl.ds(v * 16, 16)]
                bkt = jax.lax.shift_right_logical(row * 13108, 25)
                lrow = row - bkt * BR
                packed = jnp.bitwise_or(col, jax.lax.shift_left(lrow, 14))
                rank = jnp.zeros((16,), jnp.int32)
                curv = jnp.zeros((16,), jnp.int32)
                new = []
                for b in range(NB):
                    m = bkt == b
                    mi = m.astype(jnp.int32)
                    rank = rank + jnp.where(m, plsc.cumsum(mi) - mi, 0)
                    curv = curv + jnp.where(m, curs[b], 0)
                    pc = plsc.all_reduce_population_count(m)
                    new.append(curs[b] + jnp.max(pc))
                dest = bkt * CAP + curv + rank
                plsc.store_scatter(binflat, [dest], packed)
                return tuple(new)
            return lax.fori_loop(0, sl // 16, vloop, curs)

        z = jnp.int32(0)
        curs = lax.fori_loop(0, STG, stage_body, (z, z, z, z))
        for b in range(NB):
            cntbuf[b, :] = jnp.full((16,), 1, jnp.int32) * curs[b]
        pltpu.sync_copy(binflat, bins.at[pl.ds(wid * NB * CAP, NB * CAP)])
        pltpu.sync_copy(cntbuf, counts.at[pl.ds(wid * NB, NB)])

    return binner


def _make_spmm512():
    mesh = plsc.VectorSubcoreMesh(core_axis_name="c", subcore_axis_name="s")
    out_type = jax.ShapeDtypeStruct((4, NP, 128), jnp.float32)
    scratch = [
        pltpu.VMEM((SCH * KX,), jnp.int32),
        pltpu.VMEM((SCH, KX), jnp.int32),
        pltpu.VMEM((SCH, KX), jnp.int32),
        pltpu.VMEM((1, 16), jnp.int32),
        pltpu.VMEM((KX, 512), jnp.float32),
        pltpu.VMEM((KX, 512), jnp.float32),
        pltpu.VMEM((KX, 128), jnp.float32),
        pltpu.VMEM((KX, 128), jnp.float32),
        pltpu.VMEM_SHARED((BR, 128), jnp.float32),
        pltpu.VMEM_SHARED((BR, 128), jnp.float32),
        pltpu.VMEM_SHARED((BR, 128), jnp.float32),
        pltpu.VMEM_SHARED((BR, 128), jnp.float32),
        pltpu.SemaphoreType.DMA,
        pltpu.SemaphoreType.DMA,
    ]

    @functools.partial(pl.kernel, mesh=mesh, out_type=out_type,
                       scratch_types=scratch,
                       compiler_params=pltpu.CompilerParams(
                           use_tc_tiling_on_sc=True))
    def spmm(bins, counts, g, zrow_hbm, out,
             pkbuf, colu, rowu, cntb, B0, B1, C0, C1, a0, a1, a2, a3,
             sem_g, sem_s):
        accs = (a0, a1, a2, a3)
        cbufs = (C0, C1)

        def scatter4(Bx, idx):
            for f in range(4):
                Cf = cbufs[f % 2]

                @pl.when(jnp.bool_(f >= 2))
                def _():
                    pltpu.make_async_copy(
                        Cf, accs[f - 2].at[idx], sem_s).wait()

                def crow(r, _):
                    for jj in range(8):
                        Cf[r, pl.ds(jj * 16, 16)] = (
                            Bx[r, pl.ds(f * 128 + jj * 16, 16)])
                    return 0
                lax.fori_loop(0, KX, crow, 0)
                pltpu.async_copy(Cf, accs[f].at[idx], sem_s, add=True)
            pltpu.make_async_copy(C0, accs[2].at[idx], sem_s).wait()
            pltpu.make_async_copy(C1, accs[3].at[idx], sem_s).wait()
        c = lax.axis_index("c")
        s = lax.axis_index("s")
        rps = BR // 16

        for p in range(2):
            q = 2 * p + c

            def zloop(i, _):
                for f in range(4):
                    pltpu.sync_copy(
                        zrow_hbm,
                        accs[f].at[pl.ds(s * rps + i * 32, 32)])
                return 0
            lax.fori_loop(0, rps // 32, zloop, 0)
            plsc.subcore_barrier()

            for j in range(2):
                w = 2 * s + j
                binoff = (w * NB + q) * CAP
                pltpu.sync_copy(counts.at[pl.ds(w * NB + q, 1)], cntb)
                cnt = cntb[0, :][0]
                nch = ((cnt + 2 * KX - 1) // (2 * KX)) * 2
                nst = (nch + SCH - 1) // SCH

                def stage_loop(st, _):
                    pltpu.sync_copy(
                        bins.at[pl.ds(binoff + st * SCH * KX, SCH * KX)],
                        pkbuf)

                    def uloop(v, _):
                        pk = pkbuf[pl.ds(v * 16, 16)]
                        t = v // 2
                        off = (v % 2) * 16
                        colu[t, pl.ds(off, 16)] = jnp.bitwise_and(pk, 16383)
                        rowu[t, pl.ds(off, 16)] = (
                            jax.lax.shift_right_logical(pk, 14))
                        return 0
                    lax.fori_loop(0, SCH * KX // 16, uloop, 0)

                    mpairs = jnp.minimum(SCH, nch - st * SCH) // 2
                    pltpu.async_copy(g.at[colu.at[0]], B0, sem_g)

                    def pair(i, _):
                        t0 = 2 * i
                        t1 = t0 + 1
                        pltpu.async_copy(g.at[colu.at[t1]], B1, sem_g)
                        pltpu.make_async_copy(
                            g.at[colu.at[t0]], B0, sem_g).wait()
                        scatter4(B0, rowu.at[t0])

                        @pl.when(i < mpairs - 1)
                        def _():
                            pltpu.async_copy(g.at[colu.at[t0 + 2]], B0, sem_g)

                        pltpu.make_async_copy(
                            g.at[colu.at[t1]], B1, sem_g).wait()
                        scatter4(B1, rowu.at[t1])
                        return 0
                    lax.fori_loop(0, mpairs, pair, 0)
                    return 0

                @pl.when(cnt > 0)
                def _():
                    lax.fori_loop(0, nst, stage_loop, 0)

            plsc.subcore_barrier()

            def dloop(i, _):
                r0 = s * rps + i * 32
                for f in range(4):
                    pltpu.sync_copy(accs[f].at[pl.ds(r0, 32)],
                                    out.at[f].at[pl.ds(q * BR + r0, 32)])
                return 0
            lax.fori_loop(0, rps // 32, dloop, 0)
            plsc.subcore_barrier()

    return spmm


_spmm_deg = _make_spmm_deg()
_binner = _make_binner()
_spmm512 = _make_spmm512()


def _dinv_body(d0, d1, o):
    deg = d0[:, 0] + d1[:, 0] + 1.0
    o[...] = lax.rsqrt(jnp.maximum(deg, 1.0))


def _dinv_tc(d0, d1):
    return pl.pallas_call(
        _dinv_body,
        out_shape=jax.ShapeDtypeStruct((NP,), jnp.float32),
    )(d0, d1)


def _expand_body(dinv, h, w, g):
    i = pl.program_id(0)
    ridx = jax.lax.broadcasted_iota(jnp.int32, (BM, 1), 0) + i * BM
    valid = (ridx < NN).astype(jnp.float32)
    ty = valid * dinv[...][:, None] * h[...]
    u = jnp.dot(ty, w[...], preferred_element_type=jnp.float32)
    g[:, :DD] = ty
    g[:, DD:] = u


def _expand_tc(dinv, h, w):
    return pl.pallas_call(
        _expand_body,
        grid=(GB,),
        in_specs=[pl.BlockSpec((BM,), lambda i: (i,)),
                  pl.BlockSpec((BM, DD), lambda i: (i, 0)),
                  pl.BlockSpec((DD, DD), lambda i: (0, 0))],
        out_specs=pl.BlockSpec((BM, 2 * DD), lambda i: (i, 0)),
        out_shape=jax.ShapeDtypeStruct((NP, 2 * DD), jnp.float32),
    )(dinv, h, w)


def _combine_body(dinv, b, s0, s1, s2, s3, g, o):
    di = dinv[...][:, None]
    bb = b[...]
    x1a = di * (s0[...] + g[:, :128])
    x1b = di * (s1[...] + g[:, 128:256])
    ha = di * (s2[...] + g[:, 256:384]) + bb[:128][None, :]
    hb = di * (s3[...] + g[:, 384:]) + bb[128:][None, :]
    o[:, :128] = jax.nn.relu(ha) + x1a
    o[:, 128:] = jax.nn.relu(hb) + x1b


def _last_body(dinv, b, s0, s1, s2, s3, g, o):
    di = dinv[...][:, None]
    bb = b[...]
    o[:, :128] = di * (s2[...] + g[:, 256:384]) + bb[:128][None, :]
    o[:, 128:] = di * (s3[...] + g[:, 384:]) + bb[128:][None, :]


def _combine_tc(body, dinv, b, sv, g):
    def sspec(f):
        return pl.BlockSpec((BM, 128), lambda i, f=f: (f * GB + i, 0))
    return pl.pallas_call(
        body,
        grid=(GB,),
        in_specs=[pl.BlockSpec((BM,), lambda i: (i,)),
                  pl.BlockSpec((DD,), lambda i: (0,)),
                  sspec(0), sspec(1), sspec(2), sspec(3),
                  pl.BlockSpec((BM, 2 * DD), lambda i: (i, 0))],
        out_specs=pl.BlockSpec((BM, DD), lambda i: (i, 0)),
        out_shape=jax.ShapeDtypeStruct((NP, DD), jnp.float32),
    )(dinv, b, sv, sv, sv, sv, g)


@jax.jit
def _forward(x, edge_index, W0, b0, W1, b1, W2, b2, W3, b3):
    src = edge_index[0].astype(jnp.int32)
    dst = edge_index[1].astype(jnp.int32)
    pad = ED - 2 * NE
    rows = jnp.concatenate([src, dst, jnp.full((pad,), NN, jnp.int32)])
    cols = jnp.concatenate([dst, src, jnp.zeros((pad,), jnp.int32)])
    rows2 = rows.reshape(ED // KC, KC)
    cols2 = cols.reshape(ED // KC, KC)
    xp = jnp.pad(x, ((0, NP - NN), (0, 0)))
    ones128 = jnp.ones((NP, 128), jnp.float32)
    z128 = jnp.zeros((ZR, 128), jnp.float32)
    z512 = jnp.zeros((32, 128), jnp.float32)

    d0, d1 = _spmm_deg(rows2, cols2, z128, ones128)
    dinv = _dinv_tc(d0, d1)
    bins, counts = _binner(rows, cols)

    h = xp
    params = [(W0, b0), (W1, b1), (W2, b2), (W3, b3)]
    for l, (W, b) in enumerate(params):
        g = _expand_tc(dinv, h, W)
        sv = _spmm512(bins, counts, g, z512).reshape(4 * NP, 128)
        body = _combine_body if l < 3 else _last_body
        h = _combine_tc(body, dinv, b, sv, g)
    return h[:NN]


def kernel(x, edge_index, W0, b0, W1, b1, W2, b2, W3, b3):
    return _forward(x, edge_index, W0, b0, W1, b1, W2, b2, W3, b3)


# speedup vs baseline: 1.2596x; 1.2596x over previous
import functools

import jax
import jax.numpy as jnp
from jax import lax
from jax.experimental import pallas as pl
from jax.experimental.pallas import tpu as pltpu
from jax.experimental.pallas import tpu_sc as plsc

NN = 10000
NP = 10240
DD = 256
NE = 160000
NW = 32
K = 128
EW = 10240
T = EW // K
ED = NW * EW
ZR = 32
DR = 128
KC = 64
STG = 4
NTH = EW // STG // KC
BM = 512
GB = NP // BM


def _make_spmm(nf, F):
    mesh = plsc.VectorSubcoreMesh(core_axis_name="c", subcore_axis_name="s")
    out_type = [jax.ShapeDtypeStruct((nf * NP, F), jnp.float32),
                jax.ShapeDtypeStruct((nf * NP, F), jnp.float32)]
    scratch = [
        pltpu.VMEM((NTH, KC), jnp.int32),
        pltpu.VMEM((NTH, KC), jnp.int32),
        pltpu.VMEM((KC, F), jnp.float32),
        pltpu.VMEM((KC, F), jnp.float32),
        pltpu.VMEM((KC, F), jnp.float32),
        pltpu.VMEM((KC, F), jnp.float32),
        pltpu.VMEM_SHARED((NP, F), jnp.float32),
        pltpu.SemaphoreType.DMA,
        pltpu.SemaphoreType.DMA,
    ]

    @functools.partial(pl.kernel, mesh=mesh, out_type=out_type,
                       scratch_types=scratch,
                       compiler_params=pltpu.CompilerParams(
                           use_tc_tiling_on_sc=True))
    def spmm(rows_hbm, cols_hbm, zrow_hbm, *rest):
        gs = rest[:nf]
        out0, out1 = rest[nf], rest[nf + 1]
        colbuf, rowbuf, b0, b1, b2, b3, acc, sem_g, sem_s = rest[nf + 2:]
        bufs = (b0, b1, b2, b3)
        c = lax.axis_index("c")
        s = lax.axis_index("s")
        wid = s * 2 + c
        rps = NP // 16
        for f in range(nf):
            def zloop(i, _):
                pltpu.sync_copy(zrow_hbm, acc.at[pl.ds(s * rps + i * ZR, ZR)])
                return 0
            lax.fori_loop(0, rps // ZR, zloop, 0)
            plsc.subcore_barrier()

            g = gs[f]
            for h in range(STG):
                base = wid * STG * NTH + h * NTH
                pltpu.sync_copy(cols_hbm.at[pl.ds(base, NTH)], colbuf)
                pltpu.sync_copy(rows_hbm.at[pl.ds(base, NTH)], rowbuf)
                pltpu.async_copy(g.at[colbuf.at[0]], b0, sem_g)
                pltpu.async_copy(g.at[colbuf.at[1]], b1, sem_g)

                def group(i, _):
                    for b in range(4):
                        t = 4 * i + b
                        cur = bufs[b]
                        nxt = bufs[(b + 2) % 4]

                        @pl.when(t >= 2)
                        def _():
                            pltpu.make_async_copy(
                                nxt, acc.at[rowbuf.at[t - 2]], sem_s).wait()

                        @pl.when(t + 2 < NTH)
                        def _():
                            pltpu.async_copy(
                                g.at[colbuf.at[t + 2]], nxt, sem_g)

                        pltpu.make_async_copy(
                            g.at[colbuf.at[t]], cur, sem_g).wait()
                        pltpu.async_copy(
                            cur, acc.at[rowbuf.at[t]], sem_s, add=True)
                    return 0
                lax.fori_loop(0, NTH // 4, group, 0)
                pltpu.make_async_copy(
                    bufs[(NTH - 2) % 4],
                    acc.at[rowbuf.at[NTH - 2]], sem_s).wait()
                pltpu.make_async_copy(
                    bufs[(NTH - 1) % 4],
                    acc.at[rowbuf.at[NTH - 1]], sem_s).wait()
            plsc.subcore_barrier()

            def dloop(i, _):
                r0 = s * rps + i * DR

                @pl.when(c == 0)
                def _():
                    pltpu.sync_copy(acc.at[pl.ds(r0, DR)],
                                    out0.at[pl.ds(f * NP + r0, DR)])

                @pl.when(c == 1)
                def _():
                    pltpu.sync_copy(acc.at[pl.ds(r0, DR)],
                                    out1.at[pl.ds(f * NP + r0, DR)])
                return 0
            lax.fori_loop(0, rps // DR, dloop, 0)
            plsc.subcore_barrier()

    return spmm


_spmm_deg = _make_spmm(1, 128)
_spmm4 = _make_spmm(4, 128)
_spmm2 = _make_spmm(2, 128)


def _dinv_body(d0, d1, o):
    deg = d0[:, 0] + d1[:, 0] + 1.0
    o[...] = lax.rsqrt(jnp.maximum(deg, 1.0))


def _dinv_tc(d0, d1):
    return pl.pallas_call(
        _dinv_body,
        out_shape=jax.ShapeDtypeStruct((NP,), jnp.float32),
    )(d0, d1)


def _expand_body(dinv, h, w, g0, g1, g2, g3):
    ty = dinv[...][:, None] * h[...]
    u = jnp.dot(ty, w[...], preferred_element_type=jnp.float32)
    g0[...] = ty[:, :128]
    g1[...] = ty[:, 128:]
    g2[...] = u[:, :128]
    g3[...] = u[:, 128:]


def _expand_tc(dinv, h, w):
    gspec = pl.BlockSpec((BM, 128), lambda i: (i, 0))
    return pl.pallas_call(
        _expand_body,
        grid=(GB,),
        in_specs=[pl.BlockSpec((BM,), lambda i: (i,)),
                  pl.BlockSpec((BM, DD), lambda i: (i, 0)),
                  pl.BlockSpec((DD, DD), lambda i: (0, 0))],
        out_specs=[gspec, gspec, gspec, gspec],
        out_shape=[jax.ShapeDtypeStruct((NP, 128), jnp.float32)] * 4,
    )(dinv, h, w)


def _combine_body(dinv, b, s00, s01, s02, s03, s10, s11, s12, s13,
                  g0, g1, g2, g3, o):
    di = dinv[...][:, None]
    bb = b[...]
    x1a = di * (s00[...] + s10[...] + g0[...])
    x1b = di * (s01[...] + s11[...] + g1[...])
    ha = di * (s02[...] + s12[...] + g2[...]) + bb[:128][None, :]
    hb = di * (s03[...] + s13[...] + g3[...]) + bb[128:][None, :]
    o[:, :128] = jax.nn.relu(ha) + x1a
    o[:, 128:] = jax.nn.relu(hb) + x1b


def _combine_tc(dinv, b, s0, s1, g0, g1, g2, g3):
    def sspec(f):
        return pl.BlockSpec((BM, 128), lambda i, f=f: (f * GB + i, 0))
    gspec = pl.BlockSpec((BM, 128), lambda i: (i, 0))
    return pl.pallas_call(
        _combine_body,
        grid=(GB,),
        in_specs=[pl.BlockSpec((BM,), lambda i: (i,)),
                  pl.BlockSpec((DD,), lambda i: (0,)),
                  sspec(0), sspec(1), sspec(2), sspec(3),
                  sspec(0), sspec(1), sspec(2), sspec(3),
                  gspec, gspec, gspec, gspec],
        out_specs=pl.BlockSpec((BM, DD), lambda i: (i, 0)),
        out_shape=jax.ShapeDtypeStruct((NP, DD), jnp.float32),
    )(dinv, b, s0, s0, s0, s0, s1, s1, s1, s1, g0, g1, g2, g3)


def _last_body(dinv, b, s00, s01, s10, s11, g2, g3, o):
    di = dinv[...][:, None]
    bb = b[...]
    o[:, :128] = di * (s00[...] + s10[...] + g2[...]) + bb[:128][None, :]
    o[:, 128:] = di * (s01[...] + s11[...] + g3[...]) + bb[128:][None, :]


def _last_tc(dinv, b, s0, s1, g2, g3):
    def sspec(f):
        return pl.BlockSpec((BM, 128), lambda i, f=f: (f * GB + i, 0))
    gspec = pl.BlockSpec((BM, 128), lambda i: (i, 0))
    return pl.pallas_call(
        _last_body,
        grid=(GB,),
        in_specs=[pl.BlockSpec((BM,), lambda i: (i,)),
                  pl.BlockSpec((DD,), lambda i: (0,)),
                  sspec(0), sspec(1), sspec(0), sspec(1),
                  gspec, gspec],
        out_specs=pl.BlockSpec((BM, DD), lambda i: (i, 0)),
        out_shape=jax.ShapeDtypeStruct((NP, DD), jnp.float32),
    )(dinv, b, s0, s0, s1, s1, g2, g3)


@jax.jit
def _forward(x, edge_index, W0, b0, W1, b1, W2, b2, W3, b3):
    src = edge_index[0].astype(jnp.int32)
    dst = edge_index[1].astype(jnp.int32)
    pad = ED - 2 * NE
    rows = jnp.concatenate([src, dst, jnp.full((pad,), NN, jnp.int32)])
    cols = jnp.concatenate([dst, src, jnp.zeros((pad,), jnp.int32)])
    rows2 = rows.reshape(ED // KC, KC)
    cols2 = cols.reshape(ED // KC, KC)
    xp = jnp.pad(x, ((0, NP - NN), (0, 0)))
    ones128 = jnp.ones((NP, 128), jnp.float32)
    z128 = jnp.zeros((ZR, 128), jnp.float32)

    d0, d1 = _spmm_deg(rows2, cols2, z128, ones128)
    dinv = _dinv_tc(d0, d1)

    h = xp
    params = [(W0, b0), (W1, b1), (W2, b2), (W3, b3)]
    for l, (W, b) in enumerate(params):
        g0, g1, g2, g3 = _expand_tc(dinv, h, W)
        if l < 3:
            s0, s1 = _spmm4(rows2, cols2, z128, g0, g1, g2, g3)
            h = _combine_tc(dinv, b, s0, s1, g0, g1, g2, g3)
        else:
            s0, s1 = _spmm2(rows2, cols2, z128, g2, g3)
            h = _last_tc(dinv, b, s0, s1, g2, g3)
    return h[:NN]


def kernel(x, edge_index, W0, b0, W1, b1, W2, b2, W3, b3):
    return _forward(x, edge_index, W0, b0, W1, b1, W2, b2, W3, b3)


# speedup vs baseline: 1.2865x; 1.0214x over previous
import functools

import jax
import jax.numpy as jnp
from jax import lax
from jax.experimental import pallas as pl
from jax.experimental.pallas import tpu as pltpu
from jax.experimental.pallas import tpu_sc as plsc

NN = 10000
NP = 10240
DD = 256
NE = 160000
NW = 32
K = 128
EW = 10240
T = EW // K
ED = NW * EW
ZR = 32
DR = 128
KC = 64
STG = 4
NTH = EW // STG // KC
BM = 512
GB = NP // BM


def _make_spmm(nf, F):
    mesh = plsc.VectorSubcoreMesh(core_axis_name="c", subcore_axis_name="s")
    out_type = [jax.ShapeDtypeStruct((nf * NP, F), jnp.float32),
                jax.ShapeDtypeStruct((nf * NP, F), jnp.float32)]
    scratch = [
        pltpu.VMEM((NTH, KC), jnp.int32),
        pltpu.VMEM((NTH, KC), jnp.int32),
        pltpu.VMEM((KC, F), jnp.float32),
        pltpu.VMEM((KC, F), jnp.float32),
        pltpu.VMEM((KC, F), jnp.float32),
        pltpu.VMEM((KC, F), jnp.float32),
        pltpu.VMEM_SHARED((NP, F), jnp.float32),
        pltpu.SemaphoreType.DMA,
        pltpu.SemaphoreType.DMA,
    ]

    @functools.partial(pl.kernel, mesh=mesh, out_type=out_type,
                       scratch_types=scratch,
                       compiler_params=pltpu.CompilerParams(
                           use_tc_tiling_on_sc=True))
    def spmm(rows_hbm, cols_hbm, zrow_hbm, *rest):
        gs = rest[:nf]
        out0, out1 = rest[nf], rest[nf + 1]
        colbuf, rowbuf, b0, b1, b2, b3, acc, sem_g, sem_s = rest[nf + 2:]
        bufs = (b0, b1, b2, b3)
        c = lax.axis_index("c")
        s = lax.axis_index("s")
        wid = s * 2 + c
        rps = NP // 16
        for f in range(nf):
            def zloop(i, _):
                pltpu.sync_copy(zrow_hbm, acc.at[pl.ds(s * rps + i * ZR, ZR)])
                return 0
            lax.fori_loop(0, rps // ZR, zloop, 0)
            plsc.subcore_barrier()

            g = gs[f]
            for h in range(STG):
                base = wid * STG * NTH + h * NTH
                pltpu.sync_copy(cols_hbm.at[pl.ds(base, NTH)], colbuf)
                pltpu.sync_copy(rows_hbm.at[pl.ds(base, NTH)], rowbuf)
                pltpu.async_copy(g.at[colbuf.at[0]], b0, sem_g)
                pltpu.async_copy(g.at[colbuf.at[1]], b1, sem_g)

                def group(i, _):
                    for b in range(4):
                        t = 4 * i + b
                        cur = bufs[b]
                        nxt = bufs[(b + 2) % 4]

                        @pl.when(t >= 2)
                        def _():
                            pltpu.make_async_copy(
                                nxt, acc.at[rowbuf.at[t - 2]], sem_s).wait()

                        @pl.when(t + 2 < NTH)
                        def _():
                            pltpu.async_copy(
                                g.at[colbuf.at[t + 2]], nxt, sem_g)

                        pltpu.make_async_copy(
                            g.at[colbuf.at[t]], cur, sem_g).wait()
                        pltpu.async_copy(
                            cur, acc.at[rowbuf.at[t]], sem_s, add=True)
                    return 0
                lax.fori_loop(0, NTH // 4, group, 0)
                pltpu.make_async_copy(
                    bufs[(NTH - 2) % 4],
                    acc.at[rowbuf.at[NTH - 2]], sem_s).wait()
                pltpu.make_async_copy(
                    bufs[(NTH - 1) % 4],
                    acc.at[rowbuf.at[NTH - 1]], sem_s).wait()
            plsc.subcore_barrier()

            def dloop(i, _):
                r0 = s * rps + i * DR

                @pl.when(c == 0)
                def _():
                    pltpu.sync_copy(acc.at[pl.ds(r0, DR)],
                                    out0.at[pl.ds(f * NP + r0, DR)])

                @pl.when(c == 1)
                def _():
                    pltpu.sync_copy(acc.at[pl.ds(r0, DR)],
                                    out1.at[pl.ds(f * NP + r0, DR)])
                return 0
            lax.fori_loop(0, rps // DR, dloop, 0)
            plsc.subcore_barrier()

    return spmm


def _make_deg():
    mesh = plsc.VectorSubcoreMesh(core_axis_name="c", subcore_axis_name="s")
    out_type = [jax.ShapeDtypeStruct((NP, 128), jnp.float32),
                jax.ShapeDtypeStruct((NP, 128), jnp.float32)]
    scratch = [
        pltpu.VMEM((NTH, KC), jnp.int32),
        pltpu.VMEM((KC, 128), jnp.float32),
        pltpu.VMEM_SHARED((NP, 128), jnp.float32),
        pltpu.SemaphoreType.DMA,
    ]

    @functools.partial(pl.kernel, mesh=mesh, out_type=out_type,
                       scratch_types=scratch,
                       compiler_params=pltpu.CompilerParams(
                           use_tc_tiling_on_sc=True))
    def deg(rows_hbm, zrow_hbm, ones_hbm, out0, out1,
            rowbuf, onesb, acc, sem_s):
        c = lax.axis_index("c")
        s = lax.axis_index("s")
        wid = s * 2 + c
        rps = NP // 16
        pltpu.sync_copy(ones_hbm, onesb)

        def zloop(i, _):
            pltpu.sync_copy(zrow_hbm, acc.at[pl.ds(s * rps + i * ZR, ZR)])
            return 0
        lax.fori_loop(0, rps // ZR, zloop, 0)
        plsc.subcore_barrier()

        for h in range(STG):
            base = wid * STG * NTH + h * NTH
            pltpu.sync_copy(rows_hbm.at[pl.ds(base, NTH)], rowbuf)
            pltpu.async_copy(onesb, acc.at[rowbuf.at[0]], sem_s, add=True)
            pltpu.async_copy(onesb, acc.at[rowbuf.at[1]], sem_s, add=True)

            def eloop(t, _):
                pltpu.make_async_copy(
                    onesb, acc.at[rowbuf.at[t]], sem_s).wait()
                pltpu.async_copy(
                    onesb, acc.at[rowbuf.at[t + 2]], sem_s, add=True)
                return 0
            lax.fori_loop(0, NTH - 2, eloop, 0)
            pltpu.make_async_copy(
                onesb, acc.at[rowbuf.at[NTH - 2]], sem_s).wait()
            pltpu.make_async_copy(
                onesb, acc.at[rowbuf.at[NTH - 1]], sem_s).wait()
        plsc.subcore_barrier()

        def dloop(i, _):
            r0 = s * rps + i * DR

            @pl.when(c == 0)
            def _():
                pltpu.sync_copy(acc.at[pl.ds(r0, DR)], out0.at[pl.ds(r0, DR)])

            @pl.when(c == 1)
            def _():
                pltpu.sync_copy(acc.at[pl.ds(r0, DR)], out1.at[pl.ds(r0, DR)])
            return 0
        lax.fori_loop(0, rps // DR, dloop, 0)

    return deg


_deg_kernel = _make_deg()

_spmm_deg = _make_spmm(1, 128)
_spmm4 = _make_spmm(4, 128)
_spmm2 = _make_spmm(2, 128)


def _dinv_body(d0, d1, o):
    deg = d0[:, 0] + d1[:, 0] + 1.0
    o[...] = lax.rsqrt(jnp.maximum(deg, 1.0))


def _dinv_tc(d0, d1):
    return pl.pallas_call(
        _dinv_body,
        out_shape=jax.ShapeDtypeStruct((NP,), jnp.float32),
    )(d0, d1)


def _expand_body(dinv, h, w, g0, g1, g2, g3):
    ty = dinv[...][:, None] * h[...]
    u = jnp.dot(ty, w[...], preferred_element_type=jnp.float32)
    g0[...] = ty[:, :128]
    g1[...] = ty[:, 128:]
    g2[...] = u[:, :128]
    g3[...] = u[:, 128:]


def _expand_tc(dinv, h, w):
    gspec = pl.BlockSpec((BM, 128), lambda i: (i, 0))
    return pl.pallas_call(
        _expand_body,
        grid=(GB,),
        in_specs=[pl.BlockSpec((BM,), lambda i: (i,)),
                  pl.BlockSpec((BM, DD), lambda i: (i, 0)),
                  pl.BlockSpec((DD, DD), lambda i: (0, 0))],
        out_specs=[gspec, gspec, gspec, gspec],
        out_shape=[jax.ShapeDtypeStruct((NP, 128), jnp.float32)] * 4,
    )(dinv, h, w)


def _combine_body(dinv, b, s00, s01, s02, s03, s10, s11, s12, s13,
                  g0, g1, g2, g3, o):
    di = dinv[...][:, None]
    bb = b[...]
    x1a = di * (s00[...] + s10[...] + g0[...])
    x1b = di * (s01[...] + s11[...] + g1[...])
    ha = di * (s02[...] + s12[...] + g2[...]) + bb[:128][None, :]
    hb = di * (s03[...] + s13[...] + g3[...]) + bb[128:][None, :]
    o[:, :128] = jax.nn.relu(ha) + x1a
    o[:, 128:] = jax.nn.relu(hb) + x1b


def _combine_tc(dinv, b, s0, s1, g0, g1, g2, g3):
    def sspec(f):
        return pl.BlockSpec((BM, 128), lambda i, f=f: (f * GB + i, 0))
    gspec = pl.BlockSpec((BM, 128), lambda i: (i, 0))
    return pl.pallas_call(
        _combine_body,
        grid=(GB,),
        in_specs=[pl.BlockSpec((BM,), lambda i: (i,)),
                  pl.BlockSpec((DD,), lambda i: (0,)),
                  sspec(0), sspec(1), sspec(2), sspec(3),
                  sspec(0), sspec(1), sspec(2), sspec(3),
                  gspec, gspec, gspec, gspec],
        out_specs=pl.BlockSpec((BM, DD), lambda i: (i, 0)),
        out_shape=jax.ShapeDtypeStruct((NP, DD), jnp.float32),
    )(dinv, b, s0, s0, s0, s0, s1, s1, s1, s1, g0, g1, g2, g3)


def _last_body(dinv, b, s00, s01, s10, s11, g2, g3, o):
    di = dinv[...][:, None]
    bb = b[...]
    o[:, :128] = di * (s00[...] + s10[...] + g2[...]) + bb[:128][None, :]
    o[:, 128:] = di * (s01[...] + s11[...] + g3[...]) + bb[128:][None, :]


def _last_tc(dinv, b, s0, s1, g2, g3):
    def sspec(f):
        return pl.BlockSpec((BM, 128), lambda i, f=f: (f * GB + i, 0))
    gspec = pl.BlockSpec((BM, 128), lambda i: (i, 0))
    return pl.pallas_call(
        _last_body,
        grid=(GB,),
        in_specs=[pl.BlockSpec((BM,), lambda i: (i,)),
                  pl.BlockSpec((DD,), lambda i: (0,)),
                  sspec(0), sspec(1), sspec(0), sspec(1),
                  gspec, gspec],
        out_specs=pl.BlockSpec((BM, DD), lambda i: (i, 0)),
        out_shape=jax.ShapeDtypeStruct((NP, DD), jnp.float32),
    )(dinv, b, s0, s0, s1, s1, g2, g3)


@jax.jit
def _forward(x, edge_index, W0, b0, W1, b1, W2, b2, W3, b3):
    src = edge_index[0].astype(jnp.int32)
    dst = edge_index[1].astype(jnp.int32)
    pad = ED - 2 * NE
    rows = jnp.concatenate([src, dst, jnp.full((pad,), NN, jnp.int32)])
    cols = jnp.concatenate([dst, src, jnp.zeros((pad,), jnp.int32)])
    rows2 = rows.reshape(ED // KC, KC)
    cols2 = cols.reshape(ED // KC, KC)
    xp = jnp.pad(x, ((0, NP - NN), (0, 0)))
    ones_kc = jnp.ones((KC, 128), jnp.float32)
    z128 = jnp.zeros((ZR, 128), jnp.float32)

    d0, d1 = _deg_kernel(rows2, z128, ones_kc)
    dinv = _dinv_tc(d0, d1)

    h = xp
    params = [(W0, b0), (W1, b1), (W2, b2), (W3, b3)]
    for l, (W, b) in enumerate(params):
        g0, g1, g2, g3 = _expand_tc(dinv, h, W)
        if l < 3:
            s0, s1 = _spmm4(rows2, cols2, z128, g0, g1, g2, g3)
            h = _combine_tc(dinv, b, s0, s1, g0, g1, g2, g3)
        else:
            s0, s1 = _spmm2(rows2, cols2, z128, g2, g3)
            h = _last_tc(dinv, b, s0, s1, g2, g3)
    return h[:NN]


def kernel(x, edge_index, W0, b0, W1, b1, W2, b2, W3, b3):
    return _forward(x, edge_index, W0, b0, W1, b1, W2, b2, W3, b3)
